# baseline (speedup 1.0000x reference)
import functools

import jax
import jax.numpy as jnp
from jax import lax
from jax.experimental import pallas as pl
from jax.experimental.pallas import tpu as pltpu
from jax.experimental.pallas import tpu_sc as plsc

NE = 8
TOPK = 2
D = 2048
N = 2048
T = 256
P = ((N * TOPK + NE * (T - 1) + T - 1) // T) * T
GT = P // T

NC, NS, L = 2, 16, 16
NW = NC * NS


def _mm1_body(sp_ref, x_ref, w1_ref, o_ref):
    i = pl.program_id(0)

    @pl.when(i < sp_ref[GT])
    def _():
        h = jnp.dot(x_ref[...], w1_ref[0], preferred_element_type=jnp.float32)
        o_ref[...] = 0.5 * h * (1.0 + jax.lax.erf(h * 0.7071067811865476))


def _mm2_body(sp_ref, h_ref, w2_ref, wv_ref, o_ref):
    i = pl.program_id(0)

    @pl.when(i < sp_ref[GT])
    def _():
        o_ref[...] = jnp.dot(h_ref[...], w2_ref[0],
                             preferred_element_type=jnp.float32) * wv_ref[...]


def _grouped_ffn(x_sorted, w1, w2, w_of_pos, e_of_tile, nvalid):
    sp = jnp.concatenate([e_of_tile, nvalid[None]]).astype(jnp.int32)

    gs1 = pltpu.PrefetchScalarGridSpec(
        num_scalar_prefetch=1,
        grid=(GT,),
        in_specs=[
            pl.BlockSpec((T, D), lambda i, sp: (i, 0)),
            pl.BlockSpec((1, D, D), lambda i, sp: (sp[i], 0, 0)),
        ],
        out_specs=pl.BlockSpec((T, D), lambda i, sp: (i, 0)),
    )
    h_sorted = pl.pallas_call(
        _mm1_body, grid_spec=gs1,
        out_shape=jax.ShapeDtypeStruct((P, D), jnp.float32),
    )(sp, x_sorted, w1)

    gs2 = pltpu.PrefetchScalarGridSpec(
        num_scalar_prefetch=1,
        grid=(GT,),
        in_specs=[
            pl.BlockSpec((T, D), lambda i, sp: (i, 0)),
            pl.BlockSpec((1, D, D), lambda i, sp: (sp[i], 0, 0)),
            pl.BlockSpec((T, 1), lambda i, sp: (i, 0)),
        ],
        out_specs=pl.BlockSpec((T, D), lambda i, sp: (i, 0)),
    )
    return pl.pallas_call(
        _mm2_body, grid_spec=gs2,
        out_shape=jax.ShapeDtypeStruct((P, D), jnp.float32),
    )(sp, h_sorted, w2, w_of_pos)


TPW = N // NW
CC = 8
NCH = TPW // CC
_CU = 8

RPW = P // NW
CC2 = 24
NCH2 = RPW // CC2


def _xgather_body(tok_hbm, idx_hbm, out_hbm, idx_v, buf0, buf1,
                  semi0, semi1, semo0, semo1):
    wid = lax.axis_index("s") * NC + lax.axis_index("c")
    base = wid * RPW
    pltpu.sync_copy(idx_hbm.at[pl.ds(base, RPW)], idx_v)
    buf = (buf0, buf1)
    semi = (semi0, semi1)
    semo = (semo0, semo1)

    def start(c):
        s = c % 2
        return pltpu.async_copy(tok_hbm.at[idx_v.at[pl.ds(c * CC2, CC2)]],
                                buf[s], semi[s])

    pend = [start(0), start(1) if NCH2 > 1 else None]
    outp = [None, None]
    for c in range(NCH2):
        s = c % 2
        pend[s].wait()
        outp[s] = pltpu.async_copy(
            buf[s], out_hbm.at[pl.ds(base + c * CC2, CC2)], semo[s])
        if c + 2 < NCH2:
            outp[s].wait()
            outp[s] = None
            pend[s] = start(c + 2)
    for s in range(2):
        if outp[s] is not None:
            outp[s].wait()


def _xgather_sc(tokens, tok_of_pos):
    f = functools.partial(
        pl.kernel,
        out_type=jax.ShapeDtypeStruct((P, D), jnp.float32),
        mesh=plsc.VectorSubcoreMesh(core_axis_name="c", subcore_axis_name="s"),
        scratch_types=[
            pltpu.VMEM((RPW,), jnp.int32),
            pltpu.VMEM((CC2, D), jnp.float32),
            pltpu.VMEM((CC2, D), jnp.float32),
            pltpu.SemaphoreType.DMA,
            pltpu.SemaphoreType.DMA,
            pltpu.SemaphoreType.DMA,
            pltpu.SemaphoreType.DMA,
        ],
    )(_xgather_body)
    return f(tokens, tok_of_pos)


def _combine_body(y_hbm, ps0_hbm, ps1_hbm, out_hbm,
                  idx0_v, idx1_v, bufa0, bufa1, bufb0, bufb1,
                  semi0, semi1, semo):
    wid = lax.axis_index("s") * NC + lax.axis_index("c")
    base = wid * TPW
    pltpu.sync_copy(ps0_hbm.at[pl.ds(base, TPW)], idx0_v)
    pltpu.sync_copy(ps1_hbm.at[pl.ds(base, TPW)], idx1_v)
    bufa = (bufa0, bufa1)
    bufb = (bufb0, bufb1)
    semi = (semi0, semi1)

    def start(c):
        s = c % 2
        cpa = pltpu.async_copy(y_hbm.at[idx0_v.at[pl.ds(c * CC, CC)]],
                               bufa[s], semi[s])
        cpb = pltpu.async_copy(y_hbm.at[idx1_v.at[pl.ds(c * CC, CC)]],
                               bufb[s], semi[s])
        return cpa, cpb

    pend = start(0)
    out_pend = None
    for c in range(NCH):
        s = c % 2
        if out_pend is not None:
            out_pend.wait()
            out_pend = None
        if c + 1 < NCH:
            nxt = start(c + 1)
        pend[0].wait()
        pend[1].wait()
        if c + 1 < NCH:
            pend = nxt
        for r in range(CC):
            def add_row(k, _, r=r, s=s):
                for u in range(_CU):
                    sl = pl.ds((k * _CU + u) * L, L)
                    bufa[s][r, sl] = bufa[s][r, sl] + bufb[s][r, sl]
                return 0
            lax.fori_loop(0, D // (L * _CU), add_row, 0, unroll=False)
        out_pend = pltpu.async_copy(
            bufa[s], out_hbm.at[pl.ds(base + c * CC, CC)], semo)
    out_pend.wait()


def _combine_sc(y_sorted, ps0, ps1):
    f = functools.partial(
        pl.kernel,
        out_type=jax.ShapeDtypeStruct((N, D), jnp.float32),
        mesh=plsc.VectorSubcoreMesh(core_axis_name="c", subcore_axis_name="s"),
        scratch_types=[
            pltpu.VMEM((TPW,), jnp.int32),
            pltpu.VMEM((TPW,), jnp.int32),
            pltpu.VMEM((CC, D), jnp.float32),
            pltpu.VMEM((CC, D), jnp.float32),
            pltpu.VMEM((CC, D), jnp.float32),
            pltpu.VMEM((CC, D), jnp.float32),
            pltpu.SemaphoreType.DMA,
            pltpu.SemaphoreType.DMA,
            pltpu.SemaphoreType.DMA,
        ],
    )(_combine_body)
    return f(y_sorted, ps0, ps1)


def kernel(tokens, router_w, w1, w2):
    i32 = jnp.int32
    scores = jax.nn.softmax(tokens @ router_w.T, axis=-1)
    topw, topi = jax.lax.top_k(scores, TOPK)

    e_flat = topi.reshape(-1).astype(i32)
    onehot = (e_flat[:, None] == jnp.arange(NE, dtype=i32)[None, :]).astype(i32)
    cnt_inc = jnp.cumsum(onehot, axis=0)
    counts = cnt_inc[-1]
    rank = jnp.take_along_axis(cnt_inc, e_flat[:, None], axis=1)[:, 0] - 1
    pc = ((counts + T - 1) // T) * T
    cum_pc = jnp.cumsum(pc)
    po = cum_pc - pc
    pos = po[e_flat] + rank
    nvalid = (cum_pc[-1] // T).astype(i32)

    tok_of_pos = jnp.zeros((P,), i32).at[pos].set(jnp.arange(N * TOPK, dtype=i32) // TOPK)
    w_of_pos = jnp.zeros((P, 1), jnp.float32).at[pos, 0].set(topw.reshape(-1))

    tile_start = jnp.arange(GT, dtype=i32) * T
    e_of_tile = jnp.minimum(
        jnp.searchsorted(cum_pc, tile_start, side="right").astype(i32), NE - 1)
    e_last = e_of_tile[jnp.maximum(nvalid - 1, 0)]
    e_of_tile = jnp.where(jnp.arange(GT, dtype=i32) < nvalid, e_of_tile, e_last)

    x_sorted = _xgather_sc(tokens, tok_of_pos)

    y_sorted = _grouped_ffn(x_sorted, w1, w2, w_of_pos, e_of_tile, nvalid)

    ps = pos.reshape(N, TOPK)
    return _combine_sc(y_sorted, ps[:, 0], ps[:, 1])


# speedup vs baseline: 1.2470x; 1.2470x over previous
import functools

import jax
import jax.numpy as jnp
from jax import lax
from jax.experimental import pallas as pl
from jax.experimental.pallas import tpu as pltpu
from jax.experimental.pallas import tpu_sc as plsc

NE = 8
TOPK = 2
D = 2048
N = 2048
T = 256
P = ((N * TOPK + NE * (T - 1) + T - 1) // T) * T
GT = P // T

NC, NS, L = 2, 16, 16
NW = NC * NS


def _mm1_body(sp_ref, x_ref, w1_ref, o_ref):
    i = pl.program_id(0)

    @pl.when(i < sp_ref[GT])
    def _():
        h = jnp.dot(x_ref[...], w1_ref[0], preferred_element_type=jnp.float32)
        o_ref[...] = 0.5 * h * (1.0 + jax.lax.erf(h * 0.7071067811865476))


def _mm2_body(sp_ref, h_ref, w2_ref, wv_ref, o_ref):
    i = pl.program_id(0)

    @pl.when(i < sp_ref[GT])
    def _():
        o_ref[...] = jnp.dot(h_ref[...], w2_ref[0],
                             preferred_element_type=jnp.float32) * wv_ref[...]


def _grouped_ffn(x_sorted, w1, w2, w_of_pos, e_of_tile, nvalid):
    sp = jnp.concatenate([e_of_tile, nvalid[None]]).astype(jnp.int32)

    gs1 = pltpu.PrefetchScalarGridSpec(
        num_scalar_prefetch=1,
        grid=(GT,),
        in_specs=[
            pl.BlockSpec((T, D), lambda i, sp: (i, 0)),
            pl.BlockSpec((1, D, D), lambda i, sp: (sp[i], 0, 0)),
        ],
        out_specs=pl.BlockSpec((T, D), lambda i, sp: (i, 0)),
    )
    h_sorted = pl.pallas_call(
        _mm1_body, grid_spec=gs1,
        out_shape=jax.ShapeDtypeStruct((P, D), jnp.float32),
    )(sp, x_sorted, w1)

    gs2 = pltpu.PrefetchScalarGridSpec(
        num_scalar_prefetch=1,
        grid=(GT,),
        in_specs=[
            pl.BlockSpec((T, D), lambda i, sp: (i, 0)),
            pl.BlockSpec((1, D, D), lambda i, sp: (sp[i], 0, 0)),
            pl.BlockSpec((T, 1), lambda i, sp: (i, 0)),
        ],
        out_specs=pl.BlockSpec((T, D), lambda i, sp: (i, 0)),
    )
    return pl.pallas_call(
        _mm2_body, grid_spec=gs2,
        out_shape=jax.ShapeDtypeStruct((P, D), jnp.float32),
    )(sp, h_sorted, w2, w_of_pos)


TPW = N // NW
CC = 8
NCH = TPW // CC
_CU = 8


def _combine_body(y_hbm, psi_hbm, out_hbm,
                  idx_v, bi0, bi1, bo0, bo1, semi0, semi1, semo0, semo1):
    wid = lax.axis_index("s") * NC + lax.axis_index("c")
    base = wid * TPW
    pltpu.sync_copy(psi_hbm.at[pl.ds(base * TOPK, TPW * TOPK)], idx_v)
    bi = (bi0, bi1)
    bo = (bo0, bo1)
    semi = (semi0, semi1)
    semo = (semo0, semo1)

    def start(c):
        s = c % 2
        return pltpu.async_copy(
            y_hbm.at[idx_v.at[pl.ds(c * CC * TOPK, CC * TOPK)]], bi[s], semi[s])

    pend = [start(0), start(1) if NCH > 1 else None]
    outp = [None, None]
    for c in range(NCH):
        s = c % 2
        pend[s].wait()
        if outp[s] is not None:
            outp[s].wait()
            outp[s] = None
        for r in range(CC):
            def add_row(k, _, r=r, s=s):
                for u in range(_CU):
                    sl = pl.ds((k * _CU + u) * L, L)
                    bo[s][r, sl] = bi[s][2 * r, sl] + bi[s][2 * r + 1, sl]
                return 0
            lax.fori_loop(0, D // (L * _CU), add_row, 0, unroll=False)
        if c + 2 < NCH:
            pend[s] = start(c + 2)
        outp[s] = pltpu.async_copy(
            bo[s], out_hbm.at[pl.ds(base + c * CC, CC)], semo[s])
    for s in range(2):
        if outp[s] is not None:
            outp[s].wait()


def _combine_sc(y_sorted, psi):
    f = functools.partial(
        pl.kernel,
        out_type=jax.ShapeDtypeStruct((N, D), jnp.float32),
        mesh=plsc.VectorSubcoreMesh(core_axis_name="c", subcore_axis_name="s"),
        scratch_types=[
            pltpu.VMEM((TPW * TOPK,), jnp.int32),
            pltpu.VMEM((CC * TOPK, D), jnp.float32),
            pltpu.VMEM((CC * TOPK, D), jnp.float32),
            pltpu.VMEM((CC, D), jnp.float32),
            pltpu.VMEM((CC, D), jnp.float32),
            pltpu.SemaphoreType.DMA,
            pltpu.SemaphoreType.DMA,
            pltpu.SemaphoreType.DMA,
            pltpu.SemaphoreType.DMA,
        ],
    )(_combine_body)
    return f(y_sorted, psi)


def kernel(tokens, router_w, w1, w2):
    i32 = jnp.int32
    scores = jax.nn.softmax(tokens @ router_w.T, axis=-1)
    topw, topi = jax.lax.top_k(scores, TOPK)

    e_flat = topi.reshape(-1).astype(i32)
    onehot = (e_flat[:, None] == jnp.arange(NE, dtype=i32)[None, :]).astype(i32)
    cnt_inc = jnp.cumsum(onehot, axis=0)
    counts = cnt_inc[-1]
    rank = jnp.take_along_axis(cnt_inc, e_flat[:, None], axis=1)[:, 0] - 1
    pc = ((counts + T - 1) // T) * T
    cum_pc = jnp.cumsum(pc)
    po = cum_pc - pc
    pos = po[e_flat] + rank
    nvalid = (cum_pc[-1] // T).astype(i32)

    tok_of_pos = jnp.zeros((P,), i32).at[pos].set(jnp.arange(N * TOPK, dtype=i32) // TOPK)
    w_of_pos = jnp.zeros((P, 1), jnp.float32).at[pos, 0].set(topw.reshape(-1))

    tile_start = jnp.arange(GT, dtype=i32) * T
    e_of_tile = jnp.minimum(
        jnp.searchsorted(cum_pc, tile_start, side="right").astype(i32), NE - 1)
    e_last = e_of_tile[jnp.maximum(nvalid - 1, 0)]
    e_of_tile = jnp.where(jnp.arange(GT, dtype=i32) < nvalid, e_of_tile, e_last)

    x_sorted = tokens[tok_of_pos]

    y_sorted = _grouped_ffn(x_sorted, w1, w2, w_of_pos, e_of_tile, nvalid)

    return _combine_sc(y_sorted, pos)
